# Initial kernel scaffold; baseline (speedup 1.0000x reference)
import jax
import jax.numpy as jnp
from jax.experimental import pallas as pl


def kernel(features, user_features, user_mlp_w, user_mlp_b, conv1_weight, lin1_w, lin1_b, g1_w, g1_b, id_embedding, edge_index):
    raise NotImplementedError("write your pallas kernel here")


# speedup vs baseline: 4.7616x; 4.7616x over previous
import functools

import jax
import jax.numpy as jnp
from jax import lax
from jax.experimental import pallas as pl
from jax.experimental.pallas import tpu as pltpu
from jax.experimental.pallas import tpu_sc as plsc

N_ITEM = 7000
N_USER = 3000
N = N_ITEM + N_USER
E = 640000
D = 128
DP = 144
DI = 64
DU = 384

NC, NS = 2, 16
NW = NC * NS
EPW = E // NW
C = 80
NCHUNK = EPW // C
NROW = 624

_HI = lax.Precision.HIGHEST


def _leaky(x):
    return jnp.where(x >= 0.0, x, 0.01 * x)


def _dot(a, b):
    return lax.dot_general(a, b, (((1,), (0,)), ((), ())), precision=_HI,
                           preferred_element_type=jnp.float32)


def _user_body(uf_ref, w_ref, b_ref, o_ref):
    o_ref[...] = jnp.tanh(_dot(uf_ref[...], w_ref[...]) + b_ref[...])


def _user_mlp(uf, w, b):
    grid = 3
    rb = N_USER // grid
    return pl.pallas_call(
        _user_body,
        grid=(grid,),
        in_specs=[
            pl.BlockSpec((rb, DU), lambda i: (i, 0)),
            pl.BlockSpec((DU, D), lambda i: (0, 0)),
            pl.BlockSpec((1, D), lambda i: (0, 0)),
        ],
        out_specs=pl.BlockSpec((rb, D), lambda i: (i, 0)),
        out_shape=jax.ShapeDtypeStruct((N_USER, D), jnp.float32),
    )(uf, w, b)


def _prep_body(x_ref, cw_ref, l1w_ref, l1b_ref, id_ref, xw_ref, xhat_ref):
    x = x_ref[...]
    nrm = jnp.sqrt(jnp.sum(x * x, axis=1, keepdims=True))
    xn = x / jnp.maximum(nrm, 1e-12)
    xw_ref[...] = _dot(xn, cw_ref[...])
    t = _dot(xn, l1w_ref[...]) + l1b_ref[...]
    xhat_ref[...] = _leaky(t) + id_ref[...]


def _prep(x, cw, l1w, l1b, id_emb):
    grid = 5
    rb = N // grid
    return pl.pallas_call(
        _prep_body,
        grid=(grid,),
        in_specs=[
            pl.BlockSpec((rb, D), lambda i: (i, 0)),
            pl.BlockSpec((D, D), lambda i: (0, 0)),
            pl.BlockSpec((D, DI), lambda i: (0, 0)),
            pl.BlockSpec((1, DI), lambda i: (0, 0)),
            pl.BlockSpec((rb, DI), lambda i: (i, 0)),
        ],
        out_specs=[
            pl.BlockSpec((rb, D), lambda i: (i, 0)),
            pl.BlockSpec((rb, DI), lambda i: (i, 0)),
        ],
        out_shape=[
            jax.ShapeDtypeStruct((N, D), jnp.float32),
            jax.ShapeDtypeStruct((N, DI), jnp.float32),
        ],
    )(x, cw, l1w, l1b, id_emb)


def _edge_body(xw_hbm, src_hbm, dst_hbm, agg_hbm, den_hbm, idx_s, idx_d,
               rows_i, rows_j, u, den_local, agg_sh):
    cid = lax.axis_index("c")
    sid = lax.axis_index("s")
    wid = sid * NC + cid
    base = wid * EPW

    @pl.loop(0, C)
    def _zero(r):
        for k in range(D // 16):
            u[r, pl.ds(16 * k, 16)] = jnp.zeros((16,), jnp.float32)

    @pl.loop(0, N // 16)
    def _zden(i):
        den_local[pl.ds(16 * i, 16)] = jnp.zeros((16,), jnp.float32)

    row0 = sid * NROW
    for k in range(8):
        pltpu.sync_copy(u.at[pl.ds(0, 80)],
                        agg_sh.at[pl.ds(row0 + k * 80, 80)])
    plsc.subcore_barrier()

    @pl.loop(0, NCHUNK)
    def _chunk(c):
        off = base + c * C
        pltpu.sync_copy(src_hbm.at[pl.ds(off, C)], idx_s)
        pltpu.sync_copy(dst_hbm.at[pl.ds(off, C)], idx_d)
        pltpu.sync_copy(xw_hbm.at[idx_s], rows_j)
        pltpu.sync_copy(xw_hbm.at[idx_d], rows_i)

        for g in range(C // 16):
            rows16 = lax.iota(jnp.int32, 16) + 16 * g

            def _dot(d, acc):
                col = jnp.broadcast_to(d, (16,)).astype(jnp.int32)
                xj = plsc.load_gather(rows_j, [rows16, col])
                xi = plsc.load_gather(rows_i, [rows16, col])
                return acc + xi * _leaky(xj)

            e16 = plsc.parallel_loop(
                0, D, unroll=8, carry=jnp.zeros((16,), jnp.float32))(_dot)
            w = jnp.exp(e16)
            dst16 = idx_d[pl.ds(16 * g, 16)]
            plsc.addupdate_scatter(den_local, [dst16], w)

            @plsc.parallel_loop(0, D, unroll=8)
            def _scale(d):
                col = jnp.broadcast_to(d, (16,)).astype(jnp.int32)
                xj = plsc.load_gather(rows_j, [rows16, col])
                plsc.store_scatter(u, [rows16, col], w * xj)

        pltpu.sync_copy(u, agg_sh.at[idx_d], add=True)

    pltpu.sync_copy(den_local, den_hbm.at[wid])
    plsc.subcore_barrier()
    for k in range(8):
        pltpu.sync_copy(agg_sh.at[pl.ds(row0 + k * 80, 80)],
                        agg_hbm.at[cid, pl.ds(row0 + k * 80, 80)])


@functools.cache
def _edge_kernel_fn():
    return pl.kernel(
        _edge_body,
        out_type=[jax.ShapeDtypeStruct((NC, N, D), jnp.float32),
                  jax.ShapeDtypeStruct((NW, N), jnp.float32)],
        mesh=plsc.VectorSubcoreMesh(core_axis_name="c", subcore_axis_name="s",
                                    num_cores=NC, num_subcores=NS),
        compiler_params=pltpu.CompilerParams(needs_layout_passes=False),
        scratch_types=[
            pltpu.VMEM((C,), jnp.int32),
            pltpu.VMEM((C,), jnp.int32),
            pltpu.VMEM((C, D), jnp.float32),
            pltpu.VMEM((C, D), jnp.float32),
            pltpu.VMEM((C, D), jnp.float32),
            pltpu.VMEM((N,), jnp.float32),
            pltpu.VMEM_SHARED((N, D), jnp.float32),
        ],
    )


def _final_body(agg_ref, den_ref, xhat_ref, gw_ref, gb_ref, o_ref):
    a = agg_ref[0] + agg_ref[1]
    den = jnp.sum(den_ref[...], axis=0)[:, None]
    h = _leaky(a / (den + 1e-16))
    t = _dot(h, gw_ref[...]) + gb_ref[...] + xhat_ref[...]
    o_ref[...] = _leaky(t)


def _final(agg, den, xhat, gw, gb):
    return pl.pallas_call(
        _final_body,
        out_shape=jax.ShapeDtypeStruct((N, DI), jnp.float32),
    )(agg, den, xhat, gw, gb)


def kernel(features, user_features, user_mlp_w, user_mlp_b, conv1_weight,
           lin1_w, lin1_b, g1_w, g1_b, id_embedding, edge_index):
    user = _user_mlp(user_features, user_mlp_w, user_mlp_b.reshape(1, D))
    x = jnp.concatenate([features, user], axis=0)
    xw, xhat = _prep(x, conv1_weight, lin1_w, lin1_b.reshape(1, DI),
                     id_embedding)
    agg, den = _edge_kernel_fn()(xw, edge_index[0], edge_index[1])
    return _final(agg, den, xhat, g1_w, g1_b.reshape(1, DI))


# speedup vs baseline: 5.6978x; 1.1966x over previous
import functools

import jax
import jax.numpy as jnp
from jax import lax
from jax.experimental import pallas as pl
from jax.experimental.pallas import tpu as pltpu
from jax.experimental.pallas import tpu_sc as plsc

N_ITEM = 7000
N_USER = 3000
N = N_ITEM + N_USER
E = 640000
D = 128
DP = 144
DI = 64
DU = 384

NC, NS = 2, 16
NW = NC * NS
EPW = E // NW
C = 32
NCHUNK = EPW // C
NROW = 624

_HI = lax.Precision.HIGHEST


def _leaky(x):
    return jnp.where(x >= 0.0, x, 0.01 * x)


def _dot(a, b):
    return lax.dot_general(a, b, (((1,), (0,)), ((), ())), precision=_HI,
                           preferred_element_type=jnp.float32)


def _user_body(uf_ref, w_ref, b_ref, o_ref):
    o_ref[...] = jnp.tanh(_dot(uf_ref[...], w_ref[...]) + b_ref[...])


def _user_mlp(uf, w, b):
    grid = 3
    rb = N_USER // grid
    return pl.pallas_call(
        _user_body,
        grid=(grid,),
        in_specs=[
            pl.BlockSpec((rb, DU), lambda i: (i, 0)),
            pl.BlockSpec((DU, D), lambda i: (0, 0)),
            pl.BlockSpec((1, D), lambda i: (0, 0)),
        ],
        out_specs=pl.BlockSpec((rb, D), lambda i: (i, 0)),
        out_shape=jax.ShapeDtypeStruct((N_USER, D), jnp.float32),
    )(uf, w, b)


def _prep_body(x_ref, cw_ref, l1w_ref, l1b_ref, id_ref, xw_ref, xhat_ref):
    x = x_ref[...]
    nrm = jnp.sqrt(jnp.sum(x * x, axis=1, keepdims=True))
    xn = x / jnp.maximum(nrm, 1e-12)
    xw_ref[...] = _dot(xn, cw_ref[...])
    t = _dot(xn, l1w_ref[...]) + l1b_ref[...]
    xhat_ref[...] = _leaky(t) + id_ref[...]


def _prep(x, cw, l1w, l1b, id_emb):
    grid = 5
    rb = N // grid
    return pl.pallas_call(
        _prep_body,
        grid=(grid,),
        in_specs=[
            pl.BlockSpec((rb, D), lambda i: (i, 0)),
            pl.BlockSpec((D, D), lambda i: (0, 0)),
            pl.BlockSpec((D, DI), lambda i: (0, 0)),
            pl.BlockSpec((1, DI), lambda i: (0, 0)),
            pl.BlockSpec((rb, DI), lambda i: (i, 0)),
        ],
        out_specs=[
            pl.BlockSpec((rb, D), lambda i: (i, 0)),
            pl.BlockSpec((rb, DI), lambda i: (i, 0)),
        ],
        out_shape=[
            jax.ShapeDtypeStruct((N, D), jnp.float32),
            jax.ShapeDtypeStruct((N, DI), jnp.float32),
        ],
    )(x, cw, l1w, l1b, id_emb)


def _edge_body(xw_hbm, src_hbm, dst_hbm, agg_hbm, den_hbm,
               ixs0, ixs1, ixs2, ixs3, ixd0, ixd1, ixd2, ixd3,
               rows_i0, rows_i1, rows_j0, rows_j1, u0, u1, den_local, agg_sh,
               sis0, sis1, sis2, sis3, sid0, sid1, sid2, sid3,
               sem_i0, sem_i1, sem_j0, sem_j1, sem_u0, sem_u1):
    cid = lax.axis_index("c")
    sid = lax.axis_index("s")
    wid = sid * NC + cid

    ixs = (ixs0, ixs1, ixs2, ixs3)
    ixd = (ixd0, ixd1, ixd2, ixd3)
    sis = (sis0, sis1, sis2, sis3)
    sid_ = (sid0, sid1, sid2, sid3)
    rows_i = (rows_i0, rows_i1)
    rows_j = (rows_j0, rows_j1)
    u = (u0, u1)
    sem_i = (sem_i0, sem_i1)
    sem_j = (sem_j0, sem_j1)
    sem_u = (sem_u0, sem_u1)

    @pl.loop(0, C)
    def _zero(r):
        for k in range(D // 16):
            u0[r, pl.ds(16 * k, 16)] = jnp.zeros((16,), jnp.float32)

    @pl.loop(0, N // 16)
    def _zden(i):
        den_local[pl.ds(16 * i, 16)] = jnp.zeros((16,), jnp.float32)

    row0 = sid * NROW
    for k in range(20):
        pltpu.sync_copy(u0.at[pl.ds(0, C)],
                        agg_sh.at[pl.ds(row0 + k * C, C)])
    plsc.subcore_barrier()

    def _start_idx(c, s):
        pltpu.async_copy(src_hbm.at[wid, c], ixs[s], sis[s])
        pltpu.async_copy(dst_hbm.at[wid, c], ixd[s], sid_[s])

    def _wait_idx(c, s):
        pltpu.make_async_copy(src_hbm.at[wid, c], ixs[s], sis[s]).wait()
        pltpu.make_async_copy(dst_hbm.at[wid, c], ixd[s], sid_[s]).wait()

    def _start_gathers(s, b):
        pltpu.async_copy(xw_hbm.at[ixs[s]], rows_j[b], sem_j[b])
        pltpu.async_copy(xw_hbm.at[ixd[s]], rows_i[b], sem_i[b])

    def _wait_gathers(s, b):
        pltpu.make_async_copy(xw_hbm.at[ixs[s]], rows_j[b], sem_j[b]).wait()
        pltpu.make_async_copy(xw_hbm.at[ixd[s]], rows_i[b], sem_i[b]).wait()

    def _start_scatter(s, b):
        pltpu.async_copy(u[b], agg_sh.at[ixd[s]], sem_u[b], add=True)

    def _wait_scatter(s, b):
        pltpu.make_async_copy(u[b], agg_sh.at[ixd[s]], sem_u[b]).wait()

    def _compute(s, b):
        for g in range(C // 16):
            rows16 = lax.iota(jnp.int32, 16) + 16 * g

            def _dot(d, acc):
                col = jnp.broadcast_to(d, (16,)).astype(jnp.int32)
                xj = plsc.load_gather(rows_j[b], [rows16, col])
                xi = plsc.load_gather(rows_i[b], [rows16, col])
                return acc + xi * _leaky(xj)

            e16 = plsc.parallel_loop(
                0, D, unroll=8, carry=jnp.zeros((16,), jnp.float32))(_dot)
            w = jnp.exp(e16)
            dst16 = ixd[s][pl.ds(16 * g, 16)]
            plsc.addupdate_scatter(den_local, [dst16], w)

            @plsc.parallel_loop(0, D, unroll=8)
            def _scale(d):
                col = jnp.broadcast_to(d, (16,)).astype(jnp.int32)
                xj = plsc.load_gather(rows_j[b], [rows16, col])
                plsc.store_scatter(u[b], [rows16, col], w * xj)

    _start_idx(0, 0)
    _start_idx(1, 1)
    _wait_idx(0, 0)
    _start_gathers(0, 0)

    @pl.loop(0, NCHUNK - 1, step=4)
    def _chunk(c0):
        for sI in range(4):
            c = c0 + sI
            b = sI % 2

            @pl.when(c >= 2)
            def _():
                _wait_scatter((sI + 2) % 4, b)

            @pl.when(c + 2 < NCHUNK)
            def _():
                _start_idx(c + 2, (sI + 2) % 4)

            @pl.when(c + 1 < NCHUNK)
            def _():
                _wait_idx(c + 1, (sI + 1) % 4)
                _start_gathers((sI + 1) % 4, 1 - b)

            _wait_gathers(sI, b)
            _compute(sI, b)
            _start_scatter(sI, b)

    _wait_scatter(2, 0)
    _wait_gathers(0, 0)
    _compute(0, 0)
    _start_scatter(0, 0)

    _wait_scatter(3, 1)
    _wait_scatter(0, 0)

    pltpu.sync_copy(den_local, den_hbm.at[wid])
    plsc.subcore_barrier()
    for k in range(8):
        pltpu.sync_copy(agg_sh.at[pl.ds(row0 + k * 80, 80)],
                        agg_hbm.at[cid, pl.ds(row0 + k * 80, 80)])


@functools.cache
def _edge_kernel_fn():
    return pl.kernel(
        _edge_body,
        out_type=[jax.ShapeDtypeStruct((NC, N, D), jnp.float32),
                  jax.ShapeDtypeStruct((NW, N), jnp.float32)],
        mesh=plsc.VectorSubcoreMesh(core_axis_name="c", subcore_axis_name="s",
                                    num_cores=NC, num_subcores=NS),
        compiler_params=pltpu.CompilerParams(needs_layout_passes=False),
        scratch_types=(
            [pltpu.VMEM((C,), jnp.int32)] * 8
            + [pltpu.VMEM((C, D), jnp.float32)] * 6
            + [pltpu.VMEM((N,), jnp.float32),
               pltpu.VMEM_SHARED((N, D), jnp.float32)]
            + [pltpu.SemaphoreType.DMA] * 14
        ),
    )


def _final_body(agg_ref, den_ref, xhat_ref, gw_ref, gb_ref, o_ref):
    a = agg_ref[0] + agg_ref[1]
    den = jnp.sum(den_ref[...], axis=0)[:, None]
    h = _leaky(a / (den + 1e-16))
    t = _dot(h, gw_ref[...]) + gb_ref[...] + xhat_ref[...]
    o_ref[...] = _leaky(t)


def _final(agg, den, xhat, gw, gb):
    return pl.pallas_call(
        _final_body,
        out_shape=jax.ShapeDtypeStruct((N, DI), jnp.float32),
    )(agg, den, xhat, gw, gb)


def kernel(features, user_features, user_mlp_w, user_mlp_b, conv1_weight,
           lin1_w, lin1_b, g1_w, g1_b, id_embedding, edge_index):
    user = _user_mlp(user_features, user_mlp_w, user_mlp_b.reshape(1, D))
    x = jnp.concatenate([features, user], axis=0)
    xw, xhat = _prep(x, conv1_weight, lin1_w, lin1_b.reshape(1, DI),
                     id_embedding)
    src3 = edge_index[0].reshape(NW, NCHUNK, C)
    dst3 = edge_index[1].reshape(NW, NCHUNK, C)
    agg, den = _edge_kernel_fn()(xw, src3, dst3)
    return _final(agg, den, xhat, g1_w, g1_b.reshape(1, DI))


# speedup vs baseline: 41.9941x; 7.3703x over previous
import functools

import jax
import jax.numpy as jnp
from jax import lax
from jax.experimental import pallas as pl
from jax.experimental.pallas import tpu as pltpu
from jax.experimental.pallas import tpu_sc as plsc

N_ITEM = 7000
N_USER = 3000
N = N_ITEM + N_USER
E = 640000
D = 128
DP = 144
DI = 64
DU = 384

NC, NS = 2, 16
NW = NC * NS
EPW = E // NW
C = 32
NCHUNK = EPW // C
NROW = 624

_HI = lax.Precision.HIGHEST


def _leaky(x):
    return jnp.where(x >= 0.0, x, 0.01 * x)


def _dot(a, b):
    return lax.dot_general(a, b, (((1,), (0,)), ((), ())), precision=_HI,
                           preferred_element_type=jnp.float32)


def _user_body(uf_ref, w_ref, b_ref, o_ref):
    o_ref[...] = jnp.tanh(_dot(uf_ref[...], w_ref[...]) + b_ref[...])


def _user_mlp(uf, w, b):
    grid = 3
    rb = N_USER // grid
    return pl.pallas_call(
        _user_body,
        grid=(grid,),
        in_specs=[
            pl.BlockSpec((rb, DU), lambda i: (i, 0)),
            pl.BlockSpec((DU, D), lambda i: (0, 0)),
            pl.BlockSpec((1, D), lambda i: (0, 0)),
        ],
        out_specs=pl.BlockSpec((rb, D), lambda i: (i, 0)),
        out_shape=jax.ShapeDtypeStruct((N_USER, D), jnp.float32),
    )(uf, w, b)


def _prep_body(x_ref, cw_ref, l1w_ref, l1b_ref, id_ref, xw_ref, xhat_ref):
    x = x_ref[...]
    nrm = jnp.sqrt(jnp.sum(x * x, axis=1, keepdims=True))
    xn = x / jnp.maximum(nrm, 1e-12)
    xw_ref[...] = _dot(xn, cw_ref[...])
    t = _dot(xn, l1w_ref[...]) + l1b_ref[...]
    xhat_ref[...] = _leaky(t) + id_ref[...]


def _prep(x, cw, l1w, l1b, id_emb):
    grid = 5
    rb = N // grid
    return pl.pallas_call(
        _prep_body,
        grid=(grid,),
        in_specs=[
            pl.BlockSpec((rb, D), lambda i: (i, 0)),
            pl.BlockSpec((D, D), lambda i: (0, 0)),
            pl.BlockSpec((D, DI), lambda i: (0, 0)),
            pl.BlockSpec((1, DI), lambda i: (0, 0)),
            pl.BlockSpec((rb, DI), lambda i: (i, 0)),
        ],
        out_specs=[
            pl.BlockSpec((rb, D), lambda i: (i, 0)),
            pl.BlockSpec((rb, DI), lambda i: (i, 0)),
        ],
        out_shape=[
            jax.ShapeDtypeStruct((N, D), jnp.float32),
            jax.ShapeDtypeStruct((N, DI), jnp.float32),
        ],
    )(x, cw, l1w, l1b, id_emb)


def _edge_body(xw_hbm, src_hbm, dst_hbm, agg_hbm, den_hbm,
               ixs0, ixs1, ixs2, ixs3, ixd0, ixd1, ixd2, ixd3,
               rows_i0, rows_i1, rows_j0, rows_j1, u0, u1, ebuf, den_local,
               agg_sh,
               sis0, sis1, sis2, sis3, sid0, sid1, sid2, sid3,
               sem_i0, sem_i1, sem_j0, sem_j1, sem_u0, sem_u1):
    cid = lax.axis_index("c")
    sid = lax.axis_index("s")
    wid = sid * NC + cid

    ixs = (ixs0, ixs1, ixs2, ixs3)
    ixd = (ixd0, ixd1, ixd2, ixd3)
    sis = (sis0, sis1, sis2, sis3)
    sid_ = (sid0, sid1, sid2, sid3)
    rows_i = (rows_i0, rows_i1)
    rows_j = (rows_j0, rows_j1)
    u = (u0, u1)
    sem_i = (sem_i0, sem_i1)
    sem_j = (sem_j0, sem_j1)
    sem_u = (sem_u0, sem_u1)

    @pl.loop(0, C)
    def _zero(r):
        for k in range(D // 16):
            u0[r, pl.ds(16 * k, 16)] = jnp.zeros((16,), jnp.float32)

    @pl.loop(0, N // 16)
    def _zden(i):
        den_local[pl.ds(16 * i, 16)] = jnp.zeros((16,), jnp.float32)

    row0 = sid * NROW
    for k in range(20):
        pltpu.sync_copy(u0.at[pl.ds(0, C)],
                        agg_sh.at[pl.ds(row0 + k * C, C)])
    plsc.subcore_barrier()

    def _start_idx(c, s):
        pltpu.async_copy(src_hbm.at[wid, c], ixs[s], sis[s])
        pltpu.async_copy(dst_hbm.at[wid, c], ixd[s], sid_[s])

    def _wait_idx(c, s):
        pltpu.make_async_copy(src_hbm.at[wid, c], ixs[s], sis[s]).wait()
        pltpu.make_async_copy(dst_hbm.at[wid, c], ixd[s], sid_[s]).wait()

    def _start_gathers(s, b):
        pltpu.async_copy(xw_hbm.at[ixs[s]], rows_j[b], sem_j[b])
        pltpu.async_copy(xw_hbm.at[ixd[s]], rows_i[b], sem_i[b])

    def _wait_gathers(s, b):
        pltpu.make_async_copy(xw_hbm.at[ixs[s]], rows_j[b], sem_j[b]).wait()
        pltpu.make_async_copy(xw_hbm.at[ixd[s]], rows_i[b], sem_i[b]).wait()

    def _start_scatter(s, b):
        pltpu.async_copy(u[b], agg_sh.at[ixd[s]], sem_u[b], add=True)

    def _wait_scatter(s, b):
        pltpu.make_async_copy(u[b], agg_sh.at[ixd[s]], sem_u[b]).wait()

    def _compute(s, b):
        @plsc.parallel_loop(0, C, unroll=2)
        def _edge(e):
            xj = [rows_j[b][e, pl.ds(16 * k, 16)] for k in range(8)]
            xi = [rows_i[b][e, pl.ds(16 * k, 16)] for k in range(8)]
            p = [xi[k] * _leaky(xj[k]) for k in range(8)]
            v = ((p[0] + p[1]) + (p[2] + p[3])) + ((p[4] + p[5]) + (p[6] + p[7]))
            w = jnp.exp(jnp.broadcast_to(jnp.sum(v), (16,)))
            ebuf[e, pl.ds(0, 16)] = w
            for k in range(8):
                u[b][e, pl.ds(16 * k, 16)] = w * xj[k]

        zero16 = jnp.zeros((16,), jnp.int32)
        for g in range(C // 16):
            rows16 = lax.iota(jnp.int32, 16) + 16 * g
            w16 = plsc.load_gather(ebuf, [rows16, zero16])
            dst16 = ixd[s][pl.ds(16 * g, 16)]
            plsc.addupdate_scatter(den_local, [dst16], w16)

    _start_idx(0, 0)
    _start_idx(1, 1)
    _wait_idx(0, 0)
    _start_gathers(0, 0)

    @pl.loop(0, NCHUNK - 1, step=4)
    def _chunk(c0):
        for sI in range(4):
            c = c0 + sI
            b = sI % 2

            @pl.when(c >= 2)
            def _():
                _wait_scatter((sI + 2) % 4, b)

            @pl.when(c + 2 < NCHUNK)
            def _():
                _start_idx(c + 2, (sI + 2) % 4)

            @pl.when(c + 1 < NCHUNK)
            def _():
                _wait_idx(c + 1, (sI + 1) % 4)
                _start_gathers((sI + 1) % 4, 1 - b)

            _wait_gathers(sI, b)
            _compute(sI, b)
            _start_scatter(sI, b)

    _wait_scatter(2, 0)
    _wait_gathers(0, 0)
    _compute(0, 0)
    _start_scatter(0, 0)

    _wait_scatter(3, 1)
    _wait_scatter(0, 0)

    pltpu.sync_copy(den_local, den_hbm.at[wid])
    plsc.subcore_barrier()
    for k in range(8):
        pltpu.sync_copy(agg_sh.at[pl.ds(row0 + k * 80, 80)],
                        agg_hbm.at[cid, pl.ds(row0 + k * 80, 80)])


@functools.cache
def _edge_kernel_fn():
    return pl.kernel(
        _edge_body,
        out_type=[jax.ShapeDtypeStruct((NC, N, D), jnp.float32),
                  jax.ShapeDtypeStruct((NW, N), jnp.float32)],
        mesh=plsc.VectorSubcoreMesh(core_axis_name="c", subcore_axis_name="s",
                                    num_cores=NC, num_subcores=NS),
        compiler_params=pltpu.CompilerParams(needs_layout_passes=False),
        scratch_types=(
            [pltpu.VMEM((C,), jnp.int32)] * 8
            + [pltpu.VMEM((C, D), jnp.float32)] * 6
            + [pltpu.VMEM((C, 16), jnp.float32),
               pltpu.VMEM((N,), jnp.float32),
               pltpu.VMEM_SHARED((N, D), jnp.float32)]
            + [pltpu.SemaphoreType.DMA] * 14
        ),
    )


def _final_body(agg_ref, den_ref, xhat_ref, gw_ref, gb_ref, o_ref):
    a = agg_ref[0] + agg_ref[1]
    den = jnp.sum(den_ref[...], axis=0)[:, None]
    h = _leaky(a / (den + 1e-16))
    t = _dot(h, gw_ref[...]) + gb_ref[...] + xhat_ref[...]
    o_ref[...] = _leaky(t)


def _final(agg, den, xhat, gw, gb):
    return pl.pallas_call(
        _final_body,
        out_shape=jax.ShapeDtypeStruct((N, DI), jnp.float32),
    )(agg, den, xhat, gw, gb)


def kernel(features, user_features, user_mlp_w, user_mlp_b, conv1_weight,
           lin1_w, lin1_b, g1_w, g1_b, id_embedding, edge_index):
    user = _user_mlp(user_features, user_mlp_w, user_mlp_b.reshape(1, D))
    x = jnp.concatenate([features, user], axis=0)
    xw, xhat = _prep(x, conv1_weight, lin1_w, lin1_b.reshape(1, DI),
                     id_embedding)
    src3 = edge_index[0].reshape(NW, NCHUNK, C)
    dst3 = edge_index[1].reshape(NW, NCHUNK, C)
    agg, den = _edge_kernel_fn()(xw, src3, dst3)
    return _final(agg, den, xhat, g1_w, g1_b.reshape(1, DI))
